# baseline (speedup 1.0000x reference)
import functools

import jax
import jax.numpy as jnp
from jax import lax
from jax.experimental import pallas as pl
from jax.experimental.pallas import tpu as pltpu
from jax.experimental.pallas import tpu_sc as plsc

NUM_CORES = 2
NUM_SUBCORES = 16
LANES = 16
NW = NUM_CORES * NUM_SUBCORES


def _build_sc_call(seq: int, table_rows: int, hid: int):
    max_len = (table_rows + 1) // 2
    rows_per_w = seq // NW
    rows_per_sc = rows_per_w * NUM_SUBCORES
    win = rows_per_sc + seq - 1
    win_pad = ((win + NUM_SUBCORES * LANES - 1)
               // (NUM_SUBCORES * LANES)) * (NUM_SUBCORES * LANES)
    shard = win_pad // NUM_SUBCORES
    groups = shard // LANES

    mesh = plsc.VectorSubcoreMesh(
        core_axis_name="c", subcore_axis_name="s",
        num_cores=NUM_CORES, num_subcores=NUM_SUBCORES)

    @functools.partial(
        pl.kernel,
        out_type=jax.ShapeDtypeStruct((seq, seq, hid), jnp.float32),
        mesh=mesh,
        scratch_types=[
            pltpu.VMEM((shard,), jnp.int32),
            pltpu.VMEM((shard, hid), jnp.float32),
            pltpu.VMEM_SHARED((win_pad, hid), jnp.float32),
            pltpu.SemaphoreType.DMA,
            pltpu.SemaphoreType.DMA,
        ],
    )
    def sc_gather(table_hbm, out_hbm, idx, tbuf, shared, gsem, wsem):
        c = lax.axis_index("c")
        s = lax.axis_index("s")
        lane = lax.iota(jnp.int32, LANES)
        sc_i0 = c * rows_per_sc
        i0 = sc_i0 + s * rows_per_w

        hi = sc_i0 + (rows_per_sc - 1) + (max_len - 1)
        half_off = s * shard
        for g in range(groups):
            idx[pl.ds(g * LANES, LANES)] = jnp.maximum(
                (hi - half_off - g * LANES) - lane, 0)
        gcopy = pltpu.make_async_copy(table_hbm.at[idx], tbuf, gsem)
        gcopy.start()
        gcopy.wait()
        pltpu.sync_copy(tbuf, shared.at[pl.ds(half_off, shard), :])

        plsc.subcore_barrier()

        def row_copy(di):
            dd = (i0 - sc_i0) + di
            return pltpu.make_async_copy(
                shared.at[pl.ds((rows_per_sc - 1) - dd, seq), :],
                out_hbm.at[i0 + di],
                wsem)

        for di in range(rows_per_w):
            row_copy(di).start()
        for di in range(rows_per_w):
            row_copy(di).wait()

    return sc_gather


def kernel(x, relative_positions):
    seq = x.shape[1]
    table_rows, hid = relative_positions.shape
    call = _build_sc_call(seq, table_rows, hid)
    return call(relative_positions)


# speedup vs baseline: 1.4889x; 1.4889x over previous
import functools

import jax
import jax.numpy as jnp
from jax import lax
from jax.experimental import pallas as pl
from jax.experimental.pallas import tpu as pltpu
from jax.experimental.pallas import tpu_sc as plsc

NUM_CORES = 2
NUM_SUBCORES = 16
LANES = 16
NW = NUM_CORES * NUM_SUBCORES
CHUNKS = (512, 384, 128)


def _pad(n):
    return ((n + LANES - 1) // LANES) * LANES


def _build_sc_call(seq: int, table_rows: int, hid: int):
    max_len = (table_rows + 1) // 2
    rows_per_w = seq // NW
    assert sum(CHUNKS) == seq
    j_offs = [sum(CHUNKS[:k]) for k in range(len(CHUNKS))]
    wins = [_pad(rows_per_w + cw - 1) for cw in CHUNKS]
    buf_rows = [max(wins[0::2]), max(wins[1::2])]

    mesh = plsc.VectorSubcoreMesh(
        core_axis_name="c", subcore_axis_name="s",
        num_cores=NUM_CORES, num_subcores=NUM_SUBCORES)

    @functools.partial(
        pl.kernel,
        out_type=jax.ShapeDtypeStruct((seq, seq, hid), jnp.float32),
        mesh=mesh,
        scratch_types=[
            pltpu.VMEM((buf_rows[0],), jnp.int32),
            pltpu.VMEM((buf_rows[1],), jnp.int32),
            pltpu.VMEM((buf_rows[0], hid), jnp.float32),
            pltpu.VMEM((buf_rows[1], hid), jnp.float32),
            pltpu.SemaphoreType.DMA,
            pltpu.SemaphoreType.DMA,
            pltpu.SemaphoreType.DMA,
            pltpu.SemaphoreType.DMA,
        ],
    )
    def sc_gather(table_hbm, out_hbm, idx0, idx1, wb0, wb1, gs0, gs1, ws0, ws1):
        idx, wbuf, gsem, wsem = (idx0, idx1), (wb0, wb1), (gs0, gs1), (ws0, ws1)
        wid = lax.axis_index("s") * NUM_CORES + lax.axis_index("c")
        lane = lax.iota(jnp.int32, LANES)
        i0 = wid * rows_per_w

        def gather_window(b):
            p = b % 2
            w = wins[b]
            hi = i0 - j_offs[b] + (max_len - 1) + (rows_per_w - 1)
            for g in range(w // LANES):
                idx[p][pl.ds(g * LANES, LANES)] = jnp.maximum(
                    (hi - g * LANES) - lane, 0)
            if w == buf_rows[p]:
                src, dst = table_hbm.at[idx[p]], wbuf[p]
            else:
                src = table_hbm.at[idx[p].at[pl.ds(0, w)]]
                dst = wbuf[p].at[pl.ds(0, w), :]
            gcopy = pltpu.make_async_copy(src, dst, gsem[p])
            gcopy.start()
            gcopy.wait()

        def row_copy(b, di):
            p = b % 2
            return pltpu.make_async_copy(
                wbuf[p].at[pl.ds(rows_per_w - 1 - di, CHUNKS[b]), :],
                out_hbm.at[i0 + di, pl.ds(j_offs[b], CHUNKS[b]), :],
                wsem[p])

        n_blocks = len(CHUNKS)
        for b in range(n_blocks):
            if b >= 2:
                for di in range(rows_per_w):
                    row_copy(b - 2, di).wait()
            gather_window(b)
            for di in range(rows_per_w):
                row_copy(b, di).start()
        for b in range(max(0, n_blocks - 2), n_blocks):
            for di in range(rows_per_w):
                row_copy(b, di).wait()

    return sc_gather


def kernel(x, relative_positions):
    seq = x.shape[1]
    table_rows, hid = relative_positions.shape
    call = _build_sc_call(seq, table_rows, hid)
    return call(relative_positions)
